# initial kernel scaffold (unmeasured)
import functools

import jax
import jax.numpy as jnp
from jax import lax
from jax.experimental import pallas as pl
from jax.experimental.pallas import tpu as pltpu

N_DEV = 8
M_PER = 512
K = 4096
K_PER = 512
N = 2048


def kernel(x, w_mat):
    assert x.shape == (4096, K_PER), x.shape
    assert w_mat.shape == (K, N), w_mat.shape

    def body(x_ref, w_ref, out_ref, recv_buf, send_sems, recv_sems):
        my = lax.axis_index("i")

        barrier_sem = pltpu.get_barrier_semaphore()
        for off in range(1, N_DEV):
            dst = lax.rem(my + off, N_DEV)
            pl.semaphore_signal(
                barrier_sem, inc=1,
                device_id=(dst,), device_id_type=pl.DeviceIdType.MESH,
            )
        pl.semaphore_wait(barrier_sem, N_DEV - 1)

        rdmas = []
        for off in range(1, N_DEV):
            dst = lax.rem(my + off, N_DEV)
            rdma = pltpu.make_async_remote_copy(
                src_ref=x_ref.at[pl.ds(dst * M_PER, M_PER), :],
                dst_ref=recv_buf.at[off],
                send_sem=send_sems.at[off],
                recv_sem=recv_sems.at[off],
                device_id=(dst,),
                device_id_type=pl.DeviceIdType.MESH,
            )
            rdma.start()
            rdmas.append(rdma)

        acc = jnp.dot(
            x_ref[pl.ds(my * M_PER, M_PER), :],
            w_ref[pl.ds(my * K_PER, K_PER), :],
            preferred_element_type=jnp.float32,
        )

        for off in range(1, N_DEV):
            rdmas[off - 1].wait_recv()
            src = lax.rem(my - off + N_DEV, N_DEV)
            acc = acc + jnp.dot(
                recv_buf[off],
                w_ref[pl.ds(src * K_PER, K_PER), :],
                preferred_element_type=jnp.float32,
            )

        c = 0.7978845608028654
        out_ref[:, :] = 0.5 * acc * (1.0 + jnp.tanh(c * (acc + 0.044715 * acc * acc * acc)))

        for off in range(1, N_DEV):
            rdmas[off - 1].wait_send()

    return pl.pallas_call(
        body,
        out_shape=jax.ShapeDtypeStruct((M_PER, N), jnp.float32),
        in_specs=[
            pl.BlockSpec(memory_space=pltpu.VMEM),
            pl.BlockSpec(memory_space=pltpu.VMEM),
        ],
        out_specs=pl.BlockSpec(memory_space=pltpu.VMEM),
        scratch_shapes=[
            pltpu.VMEM((N_DEV, M_PER, K_PER), jnp.float32),
            pltpu.SemaphoreType.DMA((N_DEV,)),
            pltpu.SemaphoreType.DMA((N_DEV,)),
        ],
        compiler_params=pltpu.CompilerParams(collective_id=0),
    )(x, w_mat)


# baseline (device time: 84397 ns/iter reference)
import jax
import jax.numpy as jnp
from jax import lax
from jax.experimental import pallas as pl
from jax.experimental.pallas import tpu as pltpu

N_DEV = 8
M_PER = 512
K = 4096
K_PER = 512
N = 2048


def kernel(x, w_mat):
    assert x.shape == (4096, K_PER), x.shape
    assert w_mat.shape == (K, N), w_mat.shape

    def body(x_ref, w_ref, out_ref, recv_buf, wv, send_sems, recv_sems, w_sems):
        my = lax.axis_index("i")

        def src_at(t):
            return lax.rem(my - t + N_DEV, N_DEV)

        def w_dma(t, slot):
            return pltpu.make_async_copy(
                w_ref.at[pl.ds(src_at(t) * K_PER, K_PER), :],
                wv.at[slot],
                w_sems.at[slot],
            )

        barrier_sem = pltpu.get_barrier_semaphore()
        for off in range(1, N_DEV):
            dst = lax.rem(my + off, N_DEV)
            pl.semaphore_signal(
                barrier_sem, inc=1,
                device_id=(dst,), device_id_type=pl.DeviceIdType.MESH,
            )
        pl.semaphore_wait(barrier_sem, N_DEV - 1)

        rdmas = []
        for off in range(1, N_DEV):
            dst = lax.rem(my + off, N_DEV)
            rdma = pltpu.make_async_remote_copy(
                src_ref=x_ref.at[pl.ds(dst * M_PER, M_PER), :],
                dst_ref=recv_buf.at[off],
                send_sem=send_sems.at[off],
                recv_sem=recv_sems.at[off],
                device_id=(dst,),
                device_id_type=pl.DeviceIdType.MESH,
            )
            rdma.start()
            rdmas.append(rdma)

        w_dma(0, 0).start()
        w_dma(1, 1).start()

        for t in range(N_DEV):
            slot = t % 2
            w_dma(t, slot).wait()
            if t == 0:
                a = x_ref[pl.ds(my * M_PER, M_PER), :]
            else:
                rdmas[t - 1].wait_recv()
                a = recv_buf[t]
            partial = jnp.dot(a, wv[slot], preferred_element_type=jnp.float32)
            if t == 0:
                out_ref[:, :] = partial
            else:
                out_ref[:, :] += partial
            if t + 2 < N_DEV:
                w_dma(t + 2, slot).start()

        c = 0.7978845608028654
        y = out_ref[:, :]
        out_ref[:, :] = 0.5 * y * (1.0 + jnp.tanh(c * (y + 0.044715 * y * y * y)))

        for off in range(1, N_DEV):
            rdmas[off - 1].wait_send()

    return pl.pallas_call(
        body,
        out_shape=jax.ShapeDtypeStruct((M_PER, N), jnp.float32),
        in_specs=[
            pl.BlockSpec(memory_space=pltpu.VMEM),
            pl.BlockSpec(memory_space=pltpu.MemorySpace.HBM),
        ],
        out_specs=pl.BlockSpec(memory_space=pltpu.VMEM),
        scratch_shapes=[
            pltpu.VMEM((N_DEV, M_PER, K_PER), jnp.float32),
            pltpu.VMEM((2, K_PER, N), jnp.float32),
            pltpu.SemaphoreType.DMA((N_DEV,)),
            pltpu.SemaphoreType.DMA((N_DEV,)),
            pltpu.SemaphoreType.DMA((2,)),
        ],
        compiler_params=pltpu.CompilerParams(
            collective_id=0,
            vmem_limit_bytes=63 * 1024 * 1024,
        ),
    )(x, w_mat)


# device time: 50803 ns/iter; 1.6613x vs baseline; 1.6613x over previous
import jax
import jax.numpy as jnp
from jax import lax
from jax.experimental import pallas as pl
from jax.experimental.pallas import tpu as pltpu

N_DEV = 8
M_PER = 512
K = 4096
K_PER = 512
N = 2048


def kernel(x, w_mat):
    assert x.shape == (4096, K_PER), x.shape
    assert w_mat.shape == (K, N), w_mat.shape

    def body(x_ref, w_ref, out_ref, xb, recv_buf, wv, send_sems, recv_sems, w_sems):
        my = lax.axis_index("i")

        def src_at(t):
            return lax.rem(my - t + N_DEV, N_DEV)

        def w_dma(t, slot):
            return pltpu.make_async_copy(
                w_ref.at[pl.ds(src_at(t) * K_PER, K_PER), :],
                wv.at[slot],
                w_sems.at[slot],
            )

        barrier_sem = pltpu.get_barrier_semaphore()
        for off in range(1, N_DEV):
            dst = lax.rem(my + off, N_DEV)
            pl.semaphore_signal(
                barrier_sem, inc=1,
                device_id=(dst,), device_id_type=pl.DeviceIdType.MESH,
            )
        pl.semaphore_wait(barrier_sem, N_DEV - 1)

        xb[:, :] = x_ref[:, :].astype(jnp.bfloat16)

        rdmas = []
        for off in range(1, N_DEV):
            dst = lax.rem(my + off, N_DEV)
            rdma = pltpu.make_async_remote_copy(
                src_ref=xb.at[pl.ds(dst * M_PER, M_PER), :],
                dst_ref=recv_buf.at[off],
                send_sem=send_sems.at[off],
                recv_sem=recv_sems.at[off],
                device_id=(dst,),
                device_id_type=pl.DeviceIdType.MESH,
            )
            rdma.start()
            rdmas.append(rdma)

        w_dma(0, 0).start()
        w_dma(1, 1).start()

        for t in range(N_DEV):
            slot = t % 2
            w_dma(t, slot).wait()
            if t == 0:
                a = xb[pl.ds(my * M_PER, M_PER), :]
            else:
                rdmas[t - 1].wait_recv()
                a = recv_buf[t]
            partial = jnp.dot(
                a,
                wv[slot].astype(jnp.bfloat16),
                preferred_element_type=jnp.float32,
            )
            if t == 0:
                out_ref[:, :] = partial
            else:
                out_ref[:, :] += partial
            if t + 2 < N_DEV:
                w_dma(t + 2, slot).start()

        c = 0.7978845608028654
        y = out_ref[:, :]
        out_ref[:, :] = 0.5 * y * (1.0 + jnp.tanh(c * (y + 0.044715 * y * y * y)))

        for off in range(1, N_DEV):
            rdmas[off - 1].wait_send()

    return pl.pallas_call(
        body,
        out_shape=jax.ShapeDtypeStruct((M_PER, N), jnp.float32),
        in_specs=[
            pl.BlockSpec(memory_space=pltpu.VMEM),
            pl.BlockSpec(memory_space=pltpu.MemorySpace.HBM),
        ],
        out_specs=pl.BlockSpec(memory_space=pltpu.VMEM),
        scratch_shapes=[
            pltpu.VMEM((4096, K_PER), jnp.bfloat16),
            pltpu.VMEM((N_DEV, M_PER, K_PER), jnp.bfloat16),
            pltpu.VMEM((2, K_PER, N), jnp.float32),
            pltpu.SemaphoreType.DMA((N_DEV,)),
            pltpu.SemaphoreType.DMA((N_DEV,)),
            pltpu.SemaphoreType.DMA((2,)),
        ],
        compiler_params=pltpu.CompilerParams(
            collective_id=0,
            vmem_limit_bytes=63 * 1024 * 1024,
        ),
    )(x, w_mat)


# device time: 50332 ns/iter; 1.6768x vs baseline; 1.0094x over previous
import jax
import jax.numpy as jnp
from jax import lax
from jax.experimental import pallas as pl
from jax.experimental.pallas import tpu as pltpu

N_DEV = 8
M_PER = 512
K = 4096
K_PER = 512
N = 2048


def kernel(x, w_mat):
    assert x.shape == (4096, K_PER), x.shape
    assert w_mat.shape == (K, N), w_mat.shape

    def body(x_ref, w_ref, out_ref, xb, recv_buf, wv, send_sems, recv_sems, w_sems):
        my = lax.axis_index("i")

        def src_at(t):
            return lax.rem(my - t + N_DEV, N_DEV)

        def w_dma(t, slot):
            return pltpu.make_async_copy(
                w_ref.at[pl.ds(src_at(t) * K_PER, K_PER), :],
                wv.at[slot],
                w_sems.at[slot],
            )

        barrier_sem = pltpu.get_barrier_semaphore()
        for off in range(1, N_DEV):
            dst = lax.rem(my + off, N_DEV)
            pl.semaphore_signal(
                barrier_sem, inc=1,
                device_id=(dst,), device_id_type=pl.DeviceIdType.MESH,
            )

        w_dma(0, 0).start()
        w_dma(1, 1).start()

        xb[:, :] = x_ref[:, :].astype(jnp.bfloat16)

        pl.semaphore_wait(barrier_sem, N_DEV - 1)

        rdmas = []
        for off in range(1, N_DEV):
            dst = lax.rem(my + off, N_DEV)
            rdma = pltpu.make_async_remote_copy(
                src_ref=xb.at[pl.ds(dst * M_PER, M_PER), :],
                dst_ref=recv_buf.at[off],
                send_sem=send_sems.at[off],
                recv_sem=recv_sems.at[off],
                device_id=(dst,),
                device_id_type=pl.DeviceIdType.MESH,
            )
            rdma.start()
            rdmas.append(rdma)

        for t in range(N_DEV):
            slot = t % 2
            w_dma(t, slot).wait()
            if t == 0:
                a = xb[pl.ds(my * M_PER, M_PER), :]
            else:
                rdmas[t - 1].wait_recv()
                a = recv_buf[t]
            partial = jnp.dot(
                a,
                wv[slot].astype(jnp.bfloat16),
                preferred_element_type=jnp.float32,
            )
            if t == 0:
                out_ref[:, :] = partial
            else:
                out_ref[:, :] += partial
            if t + 2 < N_DEV:
                w_dma(t + 2, slot).start()

        c = 0.7978845608028654
        y = out_ref[:, :]
        out_ref[:, :] = 0.5 * y * (1.0 + jnp.tanh(c * (y + 0.044715 * y * y * y)))

        for off in range(1, N_DEV):
            rdmas[off - 1].wait_send()

    return pl.pallas_call(
        body,
        out_shape=jax.ShapeDtypeStruct((M_PER, N), jnp.float32),
        in_specs=[
            pl.BlockSpec(memory_space=pltpu.VMEM),
            pl.BlockSpec(memory_space=pltpu.MemorySpace.HBM),
        ],
        out_specs=pl.BlockSpec(memory_space=pltpu.VMEM),
        scratch_shapes=[
            pltpu.VMEM((4096, K_PER), jnp.bfloat16),
            pltpu.VMEM((N_DEV, M_PER, K_PER), jnp.bfloat16),
            pltpu.VMEM((2, K_PER, N), jnp.float32),
            pltpu.SemaphoreType.DMA((N_DEV,)),
            pltpu.SemaphoreType.DMA((N_DEV,)),
            pltpu.SemaphoreType.DMA((2,)),
        ],
        compiler_params=pltpu.CompilerParams(
            collective_id=0,
            vmem_limit_bytes=63 * 1024 * 1024,
        ),
    )(x, w_mat)


# device time: 22471 ns/iter; 3.7558x vs baseline; 2.2399x over previous
import os

import jax
import jax.numpy as jnp
from jax import lax
from jax.experimental import pallas as pl
from jax.experimental.pallas import tpu as pltpu

N_DEV = 8
M_PER = 512
K = 4096
K_PER = 512
N = 2048

_MODE = os.environ.get("KERNEL_MODE", "full")


def kernel(x, w_mat):
    assert x.shape == (4096, K_PER), x.shape
    assert w_mat.shape == (K, N), w_mat.shape
    comm = _MODE in ("full", "comm")
    compute = _MODE in ("full", "compute")

    def body(x_ref, w_ref, out_ref, xb, recv_buf, wv, send_sems, recv_sems, w_sems):
        my = lax.axis_index("i")

        def src_at(t):
            return lax.rem(my - t + N_DEV, N_DEV)

        def w_dma(t, slot):
            return pltpu.make_async_copy(
                w_ref.at[pl.ds(src_at(t) * K_PER, K_PER), :],
                wv.at[slot],
                w_sems.at[slot],
            )

        if comm:
            barrier_sem = pltpu.get_barrier_semaphore()
            for off in range(1, N_DEV):
                dst = lax.rem(my + off, N_DEV)
                pl.semaphore_signal(
                    barrier_sem, inc=1,
                    device_id=(dst,), device_id_type=pl.DeviceIdType.MESH,
                )

        if compute:
            w_dma(0, 0).start()
            w_dma(1, 1).start()

        xb[:, :] = x_ref[:, :].astype(jnp.bfloat16)

        if comm:
            pl.semaphore_wait(barrier_sem, N_DEV - 1)

            rdmas = []
            for off in range(1, N_DEV):
                dst = lax.rem(my + off, N_DEV)
                rdma = pltpu.make_async_remote_copy(
                    src_ref=xb.at[pl.ds(dst * M_PER, M_PER), :],
                    dst_ref=recv_buf.at[off],
                    send_sem=send_sems.at[off],
                    recv_sem=recv_sems.at[off],
                    device_id=(dst,),
                    device_id_type=pl.DeviceIdType.MESH,
                )
                rdma.start()
                rdmas.append(rdma)

        for t in range(N_DEV if compute else 0):
            slot = t % 2
            w_dma(t, slot).wait()
            if t == 0:
                a = xb[pl.ds(my * M_PER, M_PER), :]
            else:
                if comm:
                    rdmas[t - 1].wait_recv()
                a = recv_buf[t] if comm else xb[pl.ds(my * M_PER, M_PER), :]
            partial = jnp.dot(
                a,
                wv[slot].astype(jnp.bfloat16),
                preferred_element_type=jnp.float32,
            )
            if t == 0:
                out_ref[:, :] = partial
            else:
                out_ref[:, :] += partial
            if t + 2 < N_DEV:
                w_dma(t + 2, slot).start()

        if _MODE == "comm":
            out_ref[:, :] = 0.0
            for off in range(1, N_DEV):
                rdmas[off - 1].wait_recv()
                out_ref[:, :K_PER] += recv_buf[off].astype(jnp.float32)

        if compute:
            c = 0.7978845608028654
            y = out_ref[:, :]
            out_ref[:, :] = 0.5 * y * (1.0 + jnp.tanh(c * (y + 0.044715 * y * y * y)))

        if comm:
            for off in range(1, N_DEV):
                rdmas[off - 1].wait_send()

    return pl.pallas_call(
        body,
        out_shape=jax.ShapeDtypeStruct((M_PER, N), jnp.float32),
        in_specs=[
            pl.BlockSpec(memory_space=pltpu.VMEM),
            pl.BlockSpec(memory_space=pltpu.MemorySpace.HBM),
        ],
        out_specs=pl.BlockSpec(memory_space=pltpu.VMEM),
        scratch_shapes=[
            pltpu.VMEM((4096, K_PER), jnp.bfloat16),
            pltpu.VMEM((N_DEV, M_PER, K_PER), jnp.bfloat16),
            pltpu.VMEM((2, K_PER, N), jnp.float32),
            pltpu.SemaphoreType.DMA((N_DEV,)),
            pltpu.SemaphoreType.DMA((N_DEV,)),
            pltpu.SemaphoreType.DMA((2,)),
        ],
        compiler_params=pltpu.CompilerParams(
            collective_id=0 if comm else None,
            vmem_limit_bytes=63 * 1024 * 1024,
        ),
    )(x, w_mat)
